# Initial kernel scaffold; baseline (speedup 1.0000x reference)
import jax
import jax.numpy as jnp
from jax.experimental import pallas as pl


def kernel(x, edge_index, batch, roi_scaler, W1, b1, W2, b2, P1, pb1, gamma, beta, P2, pb2):
    raise NotImplementedError("write your pallas kernel here")


# speedup vs baseline: 13.1867x; 13.1867x over previous
import functools

import jax
import jax.numpy as jnp
from jax import lax
from jax.experimental import pallas as pl
from jax.experimental.pallas import tpu as pltpu
from jax.experimental.pallas import tpu_sc as plsc

N_NODES = 50000
HID = 64
QUART = 16
NQ = 4
LANES = 128
JCH = 8
N_PAD = 51200
ZROWS = 1024
RB = 2000


@functools.lru_cache(maxsize=None)
def _build_deg(rows_total):
    rows_per_core = rows_total // 2
    rows_per_tile = rows_per_core // 16
    n_chunks = rows_per_tile // JCH
    zc = N_PAD // 16
    mesh = plsc.VectorSubcoreMesh(core_axis_name="c", subcore_axis_name="s")

    @functools.partial(
        pl.kernel,
        mesh=mesh,
        out_type=jax.ShapeDtypeStruct((2, N_PAD), jnp.float32),
        compiler_params=pltpu.CompilerParams(use_tc_tiling_on_sc=False),
        scratch_types=[
            pltpu.VMEM((JCH, LANES), jnp.int32),
            pltpu.VMEM((LANES,), jnp.float32),
            pltpu.VMEM((zc,), jnp.float32),
            pltpu.VMEM_SHARED((N_PAD,), jnp.float32),
        ],
    )
    def deg_kernel(dst_hbm, out_hbm, didx, ones_v, zbuf, acc):
        c = lax.axis_index("c")
        s = lax.axis_index("s")
        one16 = jnp.ones((16,), jnp.float32)
        z16 = jnp.zeros((16,), jnp.float32)
        for i in range(LANES // 16):
            ones_v[pl.ds(i * 16, 16)] = one16

        def zr(i, carry):
            zbuf[pl.ds(i * 16, 16)] = z16
            return carry

        lax.fori_loop(0, zc // 16, zr, 0)
        pltpu.sync_copy(zbuf, acc.at[pl.ds(s * zc, zc)])
        plsc.subcore_barrier()

        def chunk(g, carry):
            rb = c * rows_per_core + s * rows_per_tile + g * JCH
            pltpu.sync_copy(dst_hbm.at[pl.ds(rb, JCH)], didx)
            for j in range(JCH):
                pltpu.sync_copy(ones_v, acc.at[didx.at[j]], add=True)
            return carry

        lax.fori_loop(0, n_chunks, chunk, 0)
        plsc.subcore_barrier()
        pltpu.sync_copy(acc.at[pl.ds(s * zc, zc)], out_hbm.at[c].at[pl.ds(s * zc, zc)])

    return deg_kernel


@functools.lru_cache(maxsize=None)
def _build_scatter(rows_total):
    rows_per_tile = rows_total // 16
    n_chunks = rows_per_tile // JCH
    zc = N_PAD // 16
    mesh = plsc.VectorSubcoreMesh(core_axis_name="c", subcore_axis_name="s")

    @functools.partial(
        pl.kernel,
        mesh=mesh,
        out_type=jax.ShapeDtypeStruct((NQ, N_PAD, QUART), jnp.float32),
        compiler_params=pltpu.CompilerParams(use_tc_tiling_on_sc=False),
        scratch_types=[
            pltpu.VMEM((JCH, LANES), jnp.int32),
            pltpu.VMEM((JCH, LANES), jnp.int32),
            pltpu.VMEM((JCH, LANES, QUART), jnp.float32),
            pltpu.VMEM((ZROWS, QUART), jnp.float32),
            pltpu.VMEM_SHARED((N_PAD, QUART), jnp.float32),
            pltpu.SemaphoreType.DMA,
        ],
    )
    def scat_kernel(src_hbm, dst_hbm, hs_hbm, out_hbm, sidx, didx, rows, zbuf, acc, sem):
        c = lax.axis_index("c")
        s = lax.axis_index("s")
        z16 = jnp.zeros((16,), jnp.float32)

        def zrow(i, carry):
            zbuf[i, pl.ds(0, 16)] = z16
            return carry

        lax.fori_loop(0, ZROWS, zrow, 0)
        zbase = s * zc

        for p in range(2):
            q = 2 * c + p
            for k in range(zc // ZROWS):
                pltpu.sync_copy(zbuf, acc.at[pl.ds(zbase + k * ZROWS, ZROWS)])
            rem = zc % ZROWS
            if rem:
                pltpu.sync_copy(zbuf.at[pl.ds(0, rem)],
                                acc.at[pl.ds(zbase + (zc // ZROWS) * ZROWS, rem)])
            plsc.subcore_barrier()

            def chunk(g, carry):
                rb = s * rows_per_tile + g * JCH
                pltpu.sync_copy(src_hbm.at[pl.ds(rb, JCH)], sidx)
                pltpu.sync_copy(dst_hbm.at[pl.ds(rb, JCH)], didx)
                cps = [pltpu.async_copy(hs_hbm.at[q].at[sidx.at[j]], rows.at[j], sem)
                       for j in range(JCH)]
                for cp in cps:
                    cp.wait()
                for j in range(JCH):
                    pltpu.sync_copy(rows.at[j], acc.at[didx.at[j]], add=True)
                return carry

            lax.fori_loop(0, n_chunks, chunk, 0)
            plsc.subcore_barrier()
            pltpu.sync_copy(acc.at[pl.ds(zbase, zc)], out_hbm.at[q].at[pl.ds(zbase, zc)])

    return scat_kernel


def _tc1_body(x_ref, dga_ref, dgb_ref, roi_ref, w_ref, hs_ref, dis_ref):
    deg = dga_ref[...] + dgb_ref[...] + 1.0
    dis = lax.rsqrt(deg)
    s = jnp.tile(roi_ref[...], (RB // 100, 1))
    h = jnp.dot(x_ref[...] * s, w_ref[...], preferred_element_type=jnp.float32)
    hs = h * dis
    for q in range(NQ):
        hs_ref[q] = hs[:, q * QUART:(q + 1) * QUART]
    dis_ref[...] = dis


def _tc1(x, dga, dgb, roi, W1):
    nb = N_NODES // RB
    return pl.pallas_call(
        _tc1_body,
        grid=(nb,),
        in_specs=[
            pl.BlockSpec((RB, 16), lambda i: (i, 0)),
            pl.BlockSpec((RB, 1), lambda i: (i, 0)),
            pl.BlockSpec((RB, 1), lambda i: (i, 0)),
            pl.BlockSpec((100, 16), lambda i: (0, 0)),
            pl.BlockSpec((16, HID), lambda i: (0, 0)),
        ],
        out_specs=[
            pl.BlockSpec((NQ, RB, QUART), lambda i: (0, i, 0)),
            pl.BlockSpec((RB, 1), lambda i: (i, 0)),
        ],
        out_shape=[
            jax.ShapeDtypeStruct((NQ, N_NODES, QUART), jnp.float32),
            jax.ShapeDtypeStruct((N_NODES, 1), jnp.float32),
        ],
    )(x, dga, dgb, roi, W1)


def _tc2_body(acc_ref, hs_ref, dis_ref, b1_ref, w2_ref, hs2_ref):
    a = jnp.concatenate(
        [acc_ref[q] + hs_ref[q] for q in range(NQ)], axis=1)
    dis = dis_ref[...]
    h = a * dis + b1_ref[...]
    h = jnp.where(h >= 0, h, 0.2 * h)
    g = jnp.dot(h, w2_ref[...], preferred_element_type=jnp.float32)
    gs = g * dis
    for q in range(NQ):
        hs2_ref[q] = gs[:, q * QUART:(q + 1) * QUART]


def _tc2(acc1, hs1, dis, b1, W2):
    nb = N_NODES // RB
    return pl.pallas_call(
        _tc2_body,
        grid=(nb,),
        in_specs=[
            pl.BlockSpec((NQ, RB, QUART), lambda i: (0, i, 0)),
            pl.BlockSpec((NQ, RB, QUART), lambda i: (0, i, 0)),
            pl.BlockSpec((RB, 1), lambda i: (i, 0)),
            pl.BlockSpec((1, HID), lambda i: (0, 0)),
            pl.BlockSpec((HID, HID), lambda i: (0, 0)),
        ],
        out_specs=pl.BlockSpec((NQ, RB, QUART), lambda i: (0, i, 0)),
        out_shape=jax.ShapeDtypeStruct((NQ, N_NODES, QUART), jnp.float32),
    )(acc1, hs1, dis, b1, W2)


def _tc3_body(acc_ref, hs_ref, dis_ref, b2_ref, z_ref):
    a = jnp.concatenate(
        [acc_ref[q] + hs_ref[q] for q in range(NQ)], axis=1)
    h = a * dis_ref[...] + b2_ref[...]
    h = jnp.where(h >= 0, h, 0.2 * h)
    z_ref[0] = jnp.max(h.reshape(RB // 100, 100, HID), axis=1)


def _tc3(acc2, hs2, dis, b2):
    nb = N_NODES // RB
    return pl.pallas_call(
        _tc3_body,
        grid=(nb,),
        in_specs=[
            pl.BlockSpec((NQ, RB, QUART), lambda i: (0, i, 0)),
            pl.BlockSpec((NQ, RB, QUART), lambda i: (0, i, 0)),
            pl.BlockSpec((RB, 1), lambda i: (i, 0)),
            pl.BlockSpec((1, HID), lambda i: (0, 0)),
        ],
        out_specs=pl.BlockSpec((1, RB // 100, HID), lambda i: (i, 0, 0)),
        out_shape=jax.ShapeDtypeStruct((nb, RB // 100, HID), jnp.float32),
    )(acc2, hs2, dis, b2)


def _tc4_body(z_ref, p1_ref, pb1_ref, g_ref, b_ref, p2_ref, pb2_ref, o_ref):
    z = jnp.dot(z_ref[...], p1_ref[...], preferred_element_type=jnp.float32) + pb1_ref[...]
    mean = jnp.mean(z, axis=0, keepdims=True)
    var = jnp.mean((z - mean) ** 2, axis=0, keepdims=True)
    zn = (z - mean) * lax.rsqrt(var + 1e-5) * g_ref[...] + b_ref[...]
    zn = jnp.where(zn >= 0, zn, 0.2 * zn)
    z2 = jnp.dot(zn, p2_ref[...], preferred_element_type=jnp.float32) + pb2_ref[...]
    nrm = jnp.sqrt(jnp.sum(z2 * z2, axis=1, keepdims=True))
    o_ref[...] = z2 / jnp.maximum(nrm, 1e-12)


def _tc4(z, P1, pb1, gamma, beta, P2, pb2):
    ng, emb = z.shape[0], P2.shape[1]
    return pl.pallas_call(
        _tc4_body,
        out_shape=jax.ShapeDtypeStruct((ng, emb), jnp.float32),
    )(z, P1, pb1, gamma, beta, P2, pb2)


def kernel(x, edge_index, batch, roi_scaler, W1, b1, W2, b2, P1, pb1, gamma, beta, P2, pb2):
    assert x.shape[0] == N_NODES
    E = edge_index.shape[1]
    rows_needed = -(-E // LANES)
    rows_total = -(-rows_needed // 256) * 256
    e_pad = rows_total * LANES - E
    src = jnp.concatenate(
        [edge_index[0], jnp.zeros((e_pad,), edge_index.dtype)]).reshape(rows_total, LANES)
    dst = jnp.concatenate(
        [edge_index[1], jnp.full((e_pad,), N_NODES, edge_index.dtype)]).reshape(rows_total, LANES)

    deg2 = _build_deg(rows_total)(dst)
    hs1, dis = _tc1(x, deg2[0].reshape(-1, 1), deg2[1].reshape(-1, 1), roi_scaler, W1)
    scat = _build_scatter(rows_total)
    acc1 = scat(src, dst, hs1)
    hs2 = _tc2(acc1, hs1, dis, b1.reshape(1, HID), W2)
    acc2 = scat(src, dst, hs2)
    z = _tc3(acc2, hs2, dis, b2.reshape(1, HID)).reshape(-1, HID)
    return _tc4(z, P1, pb1.reshape(1, -1), gamma.reshape(1, -1),
                beta.reshape(1, -1), P2, pb2.reshape(1, -1))


# speedup vs baseline: 14.6704x; 1.1125x over previous
import functools

import jax
import jax.numpy as jnp
from jax import lax
from jax.experimental import pallas as pl
from jax.experimental.pallas import tpu as pltpu
from jax.experimental.pallas import tpu_sc as plsc

N_NODES = 50000
HID = 64
QUART = 16
NQ = 4
LANES = 128
JCH = 8
JCS = 16
N_PAD = 51200
ZROWS = 1024
RB = 2000


@functools.lru_cache(maxsize=None)
def _build_deg(rows_total):
    rows_per_core = rows_total // 2
    rows_per_tile = rows_per_core // 16
    n_chunks = rows_per_tile // JCH
    zc = N_PAD // 16
    mesh = plsc.VectorSubcoreMesh(core_axis_name="c", subcore_axis_name="s")

    @functools.partial(
        pl.kernel,
        mesh=mesh,
        out_type=jax.ShapeDtypeStruct((2, N_PAD), jnp.float32),
        compiler_params=pltpu.CompilerParams(use_tc_tiling_on_sc=False),
        scratch_types=[
            pltpu.VMEM((JCH, LANES), jnp.int32),
            pltpu.VMEM((LANES,), jnp.float32),
            pltpu.VMEM((zc,), jnp.float32),
            pltpu.VMEM_SHARED((N_PAD,), jnp.float32),
        ],
    )
    def deg_kernel(dst_hbm, out_hbm, didx, ones_v, zbuf, acc):
        c = lax.axis_index("c")
        s = lax.axis_index("s")
        one16 = jnp.ones((16,), jnp.float32)
        z16 = jnp.zeros((16,), jnp.float32)
        for i in range(LANES // 16):
            ones_v[pl.ds(i * 16, 16)] = one16

        def zr(i, carry):
            zbuf[pl.ds(i * 16, 16)] = z16
            return carry

        lax.fori_loop(0, zc // 16, zr, 0)
        pltpu.sync_copy(zbuf, acc.at[pl.ds(s * zc, zc)])
        plsc.subcore_barrier()

        def chunk(g, carry):
            rb = c * rows_per_core + s * rows_per_tile + g * JCH
            pltpu.sync_copy(dst_hbm.at[pl.ds(rb, JCH)], didx)
            for j in range(JCH):
                pltpu.sync_copy(ones_v, acc.at[didx.at[j]], add=True)
            return carry

        lax.fori_loop(0, n_chunks, chunk, 0)
        plsc.subcore_barrier()
        pltpu.sync_copy(acc.at[pl.ds(s * zc, zc)], out_hbm.at[c].at[pl.ds(s * zc, zc)])

    return deg_kernel


@functools.lru_cache(maxsize=None)
def _build_scatter(rows_total):
    rows_per_tile = rows_total // 16
    n_chunks = rows_per_tile // JCS
    zc = N_PAD // 16
    mesh = plsc.VectorSubcoreMesh(core_axis_name="c", subcore_axis_name="s")

    @functools.partial(
        pl.kernel,
        mesh=mesh,
        out_type=jax.ShapeDtypeStruct((NQ, N_PAD, QUART), jnp.float32),
        compiler_params=pltpu.CompilerParams(use_tc_tiling_on_sc=False),
        scratch_types=[
            pltpu.VMEM((JCS, LANES), jnp.int32),
            pltpu.VMEM((JCS, LANES), jnp.int32),
            pltpu.VMEM((JCS, LANES, QUART), jnp.float32),
            pltpu.VMEM((ZROWS, QUART), jnp.float32),
            pltpu.VMEM_SHARED((N_PAD, QUART), jnp.float32),
            pltpu.SemaphoreType.DMA,
            pltpu.SemaphoreType.DMA,
        ],
    )
    def scat_kernel(src_hbm, dst_hbm, hs_hbm, out_hbm, sidx, didx, rows, zbuf, acc, gsem, ssem):
        c = lax.axis_index("c")
        s = lax.axis_index("s")
        z16 = jnp.zeros((16,), jnp.float32)

        def zrow(i, carry):
            zbuf[i, pl.ds(0, 16)] = z16
            return carry

        lax.fori_loop(0, ZROWS, zrow, 0)
        zbase = s * zc

        for p in range(2):
            q = 2 * c + p
            for k in range(zc // ZROWS):
                pltpu.sync_copy(zbuf, acc.at[pl.ds(zbase + k * ZROWS, ZROWS)])
            rem = zc % ZROWS
            if rem:
                pltpu.sync_copy(zbuf.at[pl.ds(0, rem)],
                                acc.at[pl.ds(zbase + (zc // ZROWS) * ZROWS, rem)])
            plsc.subcore_barrier()

            def chunk(g, carry):
                rb = s * rows_per_tile + g * JCS
                pltpu.sync_copy(src_hbm.at[pl.ds(rb, JCS)], sidx)
                pltpu.sync_copy(dst_hbm.at[pl.ds(rb, JCS)], didx)
                cps = [pltpu.async_copy(hs_hbm.at[q].at[sidx.at[j]], rows.at[j], gsem)
                       for j in range(JCS)]
                for cp in cps:
                    cp.wait()
                scs = [pltpu.async_copy(rows.at[j], acc.at[didx.at[j]], ssem, add=True)
                       for j in range(JCS)]
                for cp in scs:
                    cp.wait()
                return carry

            lax.fori_loop(0, n_chunks, chunk, 0)
            plsc.subcore_barrier()
            pltpu.sync_copy(acc.at[pl.ds(zbase, zc)], out_hbm.at[q].at[pl.ds(zbase, zc)])

    return scat_kernel


def _tc1_body(x_ref, dga_ref, dgb_ref, roi_ref, w_ref, hs_ref, dis_ref):
    deg = dga_ref[...] + dgb_ref[...] + 1.0
    dis = lax.rsqrt(deg)
    s = jnp.tile(roi_ref[...], (RB // 100, 1))
    h = jnp.dot(x_ref[...] * s, w_ref[...], preferred_element_type=jnp.float32)
    hs = h * dis
    for q in range(NQ):
        hs_ref[q] = hs[:, q * QUART:(q + 1) * QUART]
    dis_ref[...] = dis


def _tc1(x, dga, dgb, roi, W1):
    nb = N_NODES // RB
    return pl.pallas_call(
        _tc1_body,
        grid=(nb,),
        in_specs=[
            pl.BlockSpec((RB, 16), lambda i: (i, 0)),
            pl.BlockSpec((RB, 1), lambda i: (i, 0)),
            pl.BlockSpec((RB, 1), lambda i: (i, 0)),
            pl.BlockSpec((100, 16), lambda i: (0, 0)),
            pl.BlockSpec((16, HID), lambda i: (0, 0)),
        ],
        out_specs=[
            pl.BlockSpec((NQ, RB, QUART), lambda i: (0, i, 0)),
            pl.BlockSpec((RB, 1), lambda i: (i, 0)),
        ],
        out_shape=[
            jax.ShapeDtypeStruct((NQ, N_NODES, QUART), jnp.float32),
            jax.ShapeDtypeStruct((N_NODES, 1), jnp.float32),
        ],
    )(x, dga, dgb, roi, W1)


def _tc2_body(acc_ref, hs_ref, dis_ref, b1_ref, w2_ref, hs2_ref):
    a = jnp.concatenate(
        [acc_ref[q] + hs_ref[q] for q in range(NQ)], axis=1)
    dis = dis_ref[...]
    h = a * dis + b1_ref[...]
    h = jnp.where(h >= 0, h, 0.2 * h)
    g = jnp.dot(h, w2_ref[...], preferred_element_type=jnp.float32)
    gs = g * dis
    for q in range(NQ):
        hs2_ref[q] = gs[:, q * QUART:(q + 1) * QUART]


def _tc2(acc1, hs1, dis, b1, W2):
    nb = N_NODES // RB
    return pl.pallas_call(
        _tc2_body,
        grid=(nb,),
        in_specs=[
            pl.BlockSpec((NQ, RB, QUART), lambda i: (0, i, 0)),
            pl.BlockSpec((NQ, RB, QUART), lambda i: (0, i, 0)),
            pl.BlockSpec((RB, 1), lambda i: (i, 0)),
            pl.BlockSpec((1, HID), lambda i: (0, 0)),
            pl.BlockSpec((HID, HID), lambda i: (0, 0)),
        ],
        out_specs=pl.BlockSpec((NQ, RB, QUART), lambda i: (0, i, 0)),
        out_shape=jax.ShapeDtypeStruct((NQ, N_NODES, QUART), jnp.float32),
    )(acc1, hs1, dis, b1, W2)


def _tc3_body(acc_ref, hs_ref, dis_ref, b2_ref, z_ref):
    a = jnp.concatenate(
        [acc_ref[q] + hs_ref[q] for q in range(NQ)], axis=1)
    h = a * dis_ref[...] + b2_ref[...]
    h = jnp.where(h >= 0, h, 0.2 * h)
    z_ref[0] = jnp.max(h.reshape(RB // 100, 100, HID), axis=1)


def _tc3(acc2, hs2, dis, b2):
    nb = N_NODES // RB
    return pl.pallas_call(
        _tc3_body,
        grid=(nb,),
        in_specs=[
            pl.BlockSpec((NQ, RB, QUART), lambda i: (0, i, 0)),
            pl.BlockSpec((NQ, RB, QUART), lambda i: (0, i, 0)),
            pl.BlockSpec((RB, 1), lambda i: (i, 0)),
            pl.BlockSpec((1, HID), lambda i: (0, 0)),
        ],
        out_specs=pl.BlockSpec((1, RB // 100, HID), lambda i: (i, 0, 0)),
        out_shape=jax.ShapeDtypeStruct((nb, RB // 100, HID), jnp.float32),
    )(acc2, hs2, dis, b2)


def _tc4_body(z_ref, p1_ref, pb1_ref, g_ref, b_ref, p2_ref, pb2_ref, o_ref):
    z = jnp.dot(z_ref[...], p1_ref[...], preferred_element_type=jnp.float32) + pb1_ref[...]
    mean = jnp.mean(z, axis=0, keepdims=True)
    var = jnp.mean((z - mean) ** 2, axis=0, keepdims=True)
    zn = (z - mean) * lax.rsqrt(var + 1e-5) * g_ref[...] + b_ref[...]
    zn = jnp.where(zn >= 0, zn, 0.2 * zn)
    z2 = jnp.dot(zn, p2_ref[...], preferred_element_type=jnp.float32) + pb2_ref[...]
    nrm = jnp.sqrt(jnp.sum(z2 * z2, axis=1, keepdims=True))
    o_ref[...] = z2 / jnp.maximum(nrm, 1e-12)


def _tc4(z, P1, pb1, gamma, beta, P2, pb2):
    ng, emb = z.shape[0], P2.shape[1]
    return pl.pallas_call(
        _tc4_body,
        out_shape=jax.ShapeDtypeStruct((ng, emb), jnp.float32),
    )(z, P1, pb1, gamma, beta, P2, pb2)


def kernel(x, edge_index, batch, roi_scaler, W1, b1, W2, b2, P1, pb1, gamma, beta, P2, pb2):
    assert x.shape[0] == N_NODES
    E = edge_index.shape[1]
    rows_needed = -(-E // LANES)
    rows_total = -(-rows_needed // 256) * 256
    e_pad = rows_total * LANES - E
    src = jnp.concatenate(
        [edge_index[0], jnp.zeros((e_pad,), edge_index.dtype)]).reshape(rows_total, LANES)
    dst = jnp.concatenate(
        [edge_index[1], jnp.full((e_pad,), N_NODES, edge_index.dtype)]).reshape(rows_total, LANES)

    deg2 = _build_deg(rows_total)(dst)
    hs1, dis = _tc1(x, deg2[0].reshape(-1, 1), deg2[1].reshape(-1, 1), roi_scaler, W1)
    scat = _build_scatter(rows_total)
    acc1 = scat(src, dst, hs1)
    hs2 = _tc2(acc1, hs1, dis, b1.reshape(1, HID), W2)
    acc2 = scat(src, dst, hs2)
    z = _tc3(acc2, hs2, dis, b2.reshape(1, HID)).reshape(-1, HID)
    return _tc4(z, P1, pb1.reshape(1, -1), gamma.reshape(1, -1),
                beta.reshape(1, -1), P2, pb2.reshape(1, -1))


# speedup vs baseline: 16.6356x; 1.1340x over previous
import functools

import jax
import jax.numpy as jnp
from jax import lax
from jax.experimental import pallas as pl
from jax.experimental.pallas import tpu as pltpu
from jax.experimental.pallas import tpu_sc as plsc

N_NODES = 50000
HID = 64
QUART = 16
NQ = 4
LANES = 128
JCH = 8
JCS = 8
N_PAD = 51200
ZROWS = 1024
RB = 2000


@functools.lru_cache(maxsize=None)
def _build_deg(rows_total):
    rows_per_core = rows_total // 2
    rows_per_tile = rows_per_core // 16
    n_chunks = rows_per_tile // JCH
    zc = N_PAD // 16
    mesh = plsc.VectorSubcoreMesh(core_axis_name="c", subcore_axis_name="s")

    @functools.partial(
        pl.kernel,
        mesh=mesh,
        out_type=jax.ShapeDtypeStruct((2, N_PAD), jnp.float32),
        compiler_params=pltpu.CompilerParams(use_tc_tiling_on_sc=False),
        scratch_types=[
            pltpu.VMEM((JCH, LANES), jnp.int32),
            pltpu.VMEM((LANES,), jnp.float32),
            pltpu.VMEM((zc,), jnp.float32),
            pltpu.VMEM_SHARED((N_PAD,), jnp.float32),
        ],
    )
    def deg_kernel(dst_hbm, out_hbm, didx, ones_v, zbuf, acc):
        c = lax.axis_index("c")
        s = lax.axis_index("s")
        one16 = jnp.ones((16,), jnp.float32)
        z16 = jnp.zeros((16,), jnp.float32)
        for i in range(LANES // 16):
            ones_v[pl.ds(i * 16, 16)] = one16

        def zr(i, carry):
            zbuf[pl.ds(i * 16, 16)] = z16
            return carry

        lax.fori_loop(0, zc // 16, zr, 0)
        pltpu.sync_copy(zbuf, acc.at[pl.ds(s * zc, zc)])
        plsc.subcore_barrier()

        def chunk(g, carry):
            rb = c * rows_per_core + s * rows_per_tile + g * JCH
            pltpu.sync_copy(dst_hbm.at[pl.ds(rb, JCH)], didx)
            for j in range(JCH):
                pltpu.sync_copy(ones_v, acc.at[didx.at[j]], add=True)
            return carry

        lax.fori_loop(0, n_chunks, chunk, 0)
        plsc.subcore_barrier()
        pltpu.sync_copy(acc.at[pl.ds(s * zc, zc)], out_hbm.at[c].at[pl.ds(s * zc, zc)])

    return deg_kernel


@functools.lru_cache(maxsize=None)
def _build_scatter(rows_total):
    rows_per_tile = rows_total // 16
    n_chunks = rows_per_tile // JCS
    zc = N_PAD // 16
    mesh = plsc.VectorSubcoreMesh(core_axis_name="c", subcore_axis_name="s")

    @functools.partial(
        pl.kernel,
        mesh=mesh,
        out_type=jax.ShapeDtypeStruct((NQ, N_PAD, QUART), jnp.float32),
        compiler_params=pltpu.CompilerParams(use_tc_tiling_on_sc=False),
        scratch_types=[
            pltpu.VMEM((2, JCS, LANES), jnp.int32),
            pltpu.VMEM((2, JCS, LANES), jnp.int32),
            pltpu.VMEM((2, JCS, LANES, QUART), jnp.float32),
            pltpu.VMEM((ZROWS, QUART), jnp.float32),
            pltpu.VMEM_SHARED((N_PAD, QUART), jnp.float32),
            pltpu.SemaphoreType.DMA,
            pltpu.SemaphoreType.DMA,
            pltpu.SemaphoreType.DMA,
        ],
    )
    def scat_kernel(src_hbm, dst_hbm, hs_hbm, out_hbm, sidx, didx, rows, zbuf, acc,
                    gsem, ssem, isem):
        c = lax.axis_index("c")
        s = lax.axis_index("s")
        z16 = jnp.zeros((16,), jnp.float32)

        def zrow(i, carry):
            zbuf[i, pl.ds(0, 16)] = z16
            return carry

        lax.fori_loop(0, ZROWS, zrow, 0)
        zbase = s * zc
        tbase = s * rows_per_tile

        def fire_idx(g, sl):
            pltpu.async_copy(src_hbm.at[pl.ds(tbase + g * JCS, JCS)], sidx.at[sl], isem)
            pltpu.async_copy(dst_hbm.at[pl.ds(tbase + g * JCS, JCS)], didx.at[sl], isem)

        def wait_idx(g, sl):
            pltpu.make_async_copy(
                src_hbm.at[pl.ds(tbase + g * JCS, JCS)], sidx.at[sl], isem).wait()
            pltpu.make_async_copy(
                dst_hbm.at[pl.ds(tbase + g * JCS, JCS)], didx.at[sl], isem).wait()

        for p in range(2):
            q = 2 * c + p
            for k in range(zc // ZROWS):
                pltpu.sync_copy(zbuf, acc.at[pl.ds(zbase + k * ZROWS, ZROWS)])
            rem = zc % ZROWS
            if rem:
                pltpu.sync_copy(zbuf.at[pl.ds(0, rem)],
                                acc.at[pl.ds(zbase + (zc // ZROWS) * ZROWS, rem)])
            plsc.subcore_barrier()

            def fire_g(sl):
                for j in range(JCS):
                    pltpu.async_copy(hs_hbm.at[q].at[sidx.at[sl].at[j]],
                                     rows.at[sl].at[j], gsem)

            def wait_g(sl):
                for j in range(JCS):
                    pltpu.make_async_copy(hs_hbm.at[q].at[sidx.at[sl].at[j]],
                                          rows.at[sl].at[j], gsem).wait()

            def fire_s(sl):
                for j in range(JCS):
                    pltpu.async_copy(rows.at[sl].at[j], acc.at[didx.at[sl].at[j]],
                                     ssem, add=True)

            def wait_s(sl):
                for j in range(JCS):
                    pltpu.make_async_copy(rows.at[sl].at[j], acc.at[didx.at[sl].at[j]],
                                          ssem).wait()

            fire_idx(0, 0)
            wait_idx(0, 0)
            fire_g(0)

            def chunk(g, carry):
                b = g % 2
                nb = 1 - b

                @pl.when(g >= 1)
                def _():
                    wait_s(nb)

                @pl.when(g + 1 < n_chunks)
                def _():
                    fire_idx(g + 1, nb)

                wait_g(b)
                fire_s(b)

                @pl.when(g + 1 < n_chunks)
                def _():
                    wait_idx(g + 1, nb)
                    fire_g(nb)

                return carry

            lax.fori_loop(0, n_chunks, chunk, 0)
            wait_s((n_chunks - 1) % 2)
            plsc.subcore_barrier()
            pltpu.sync_copy(acc.at[pl.ds(zbase, zc)], out_hbm.at[q].at[pl.ds(zbase, zc)])

    return scat_kernel


def _tc1_body(x_ref, dga_ref, dgb_ref, roi_ref, w_ref, hs_ref, dis_ref):
    deg = dga_ref[...] + dgb_ref[...] + 1.0
    dis = lax.rsqrt(deg)
    s = jnp.tile(roi_ref[...], (RB // 100, 1))
    h = jnp.dot(x_ref[...] * s, w_ref[...], preferred_element_type=jnp.float32)
    hs = h * dis
    for q in range(NQ):
        hs_ref[q] = hs[:, q * QUART:(q + 1) * QUART]
    dis_ref[...] = dis


def _tc1(x, dga, dgb, roi, W1):
    nb = N_NODES // RB
    return pl.pallas_call(
        _tc1_body,
        grid=(nb,),
        in_specs=[
            pl.BlockSpec((RB, 16), lambda i: (i, 0)),
            pl.BlockSpec((RB, 1), lambda i: (i, 0)),
            pl.BlockSpec((RB, 1), lambda i: (i, 0)),
            pl.BlockSpec((100, 16), lambda i: (0, 0)),
            pl.BlockSpec((16, HID), lambda i: (0, 0)),
        ],
        out_specs=[
            pl.BlockSpec((NQ, RB, QUART), lambda i: (0, i, 0)),
            pl.BlockSpec((RB, 1), lambda i: (i, 0)),
        ],
        out_shape=[
            jax.ShapeDtypeStruct((NQ, N_NODES, QUART), jnp.float32),
            jax.ShapeDtypeStruct((N_NODES, 1), jnp.float32),
        ],
    )(x, dga, dgb, roi, W1)


def _tc2_body(acc_ref, hs_ref, dis_ref, b1_ref, w2_ref, hs2_ref):
    a = jnp.concatenate(
        [acc_ref[q] + hs_ref[q] for q in range(NQ)], axis=1)
    dis = dis_ref[...]
    h = a * dis + b1_ref[...]
    h = jnp.where(h >= 0, h, 0.2 * h)
    g = jnp.dot(h, w2_ref[...], preferred_element_type=jnp.float32)
    gs = g * dis
    for q in range(NQ):
        hs2_ref[q] = gs[:, q * QUART:(q + 1) * QUART]


def _tc2(acc1, hs1, dis, b1, W2):
    nb = N_NODES // RB
    return pl.pallas_call(
        _tc2_body,
        grid=(nb,),
        in_specs=[
            pl.BlockSpec((NQ, RB, QUART), lambda i: (0, i, 0)),
            pl.BlockSpec((NQ, RB, QUART), lambda i: (0, i, 0)),
            pl.BlockSpec((RB, 1), lambda i: (i, 0)),
            pl.BlockSpec((1, HID), lambda i: (0, 0)),
            pl.BlockSpec((HID, HID), lambda i: (0, 0)),
        ],
        out_specs=pl.BlockSpec((NQ, RB, QUART), lambda i: (0, i, 0)),
        out_shape=jax.ShapeDtypeStruct((NQ, N_NODES, QUART), jnp.float32),
    )(acc1, hs1, dis, b1, W2)


def _tc3_body(acc_ref, hs_ref, dis_ref, b2_ref, z_ref):
    a = jnp.concatenate(
        [acc_ref[q] + hs_ref[q] for q in range(NQ)], axis=1)
    h = a * dis_ref[...] + b2_ref[...]
    h = jnp.where(h >= 0, h, 0.2 * h)
    z_ref[0] = jnp.max(h.reshape(RB // 100, 100, HID), axis=1)


def _tc3(acc2, hs2, dis, b2):
    nb = N_NODES // RB
    return pl.pallas_call(
        _tc3_body,
        grid=(nb,),
        in_specs=[
            pl.BlockSpec((NQ, RB, QUART), lambda i: (0, i, 0)),
            pl.BlockSpec((NQ, RB, QUART), lambda i: (0, i, 0)),
            pl.BlockSpec((RB, 1), lambda i: (i, 0)),
            pl.BlockSpec((1, HID), lambda i: (0, 0)),
        ],
        out_specs=pl.BlockSpec((1, RB // 100, HID), lambda i: (i, 0, 0)),
        out_shape=jax.ShapeDtypeStruct((nb, RB // 100, HID), jnp.float32),
    )(acc2, hs2, dis, b2)


def _tc4_body(z_ref, p1_ref, pb1_ref, g_ref, b_ref, p2_ref, pb2_ref, o_ref):
    z = jnp.dot(z_ref[...], p1_ref[...], preferred_element_type=jnp.float32) + pb1_ref[...]
    mean = jnp.mean(z, axis=0, keepdims=True)
    var = jnp.mean((z - mean) ** 2, axis=0, keepdims=True)
    zn = (z - mean) * lax.rsqrt(var + 1e-5) * g_ref[...] + b_ref[...]
    zn = jnp.where(zn >= 0, zn, 0.2 * zn)
    z2 = jnp.dot(zn, p2_ref[...], preferred_element_type=jnp.float32) + pb2_ref[...]
    nrm = jnp.sqrt(jnp.sum(z2 * z2, axis=1, keepdims=True))
    o_ref[...] = z2 / jnp.maximum(nrm, 1e-12)


def _tc4(z, P1, pb1, gamma, beta, P2, pb2):
    ng, emb = z.shape[0], P2.shape[1]
    return pl.pallas_call(
        _tc4_body,
        out_shape=jax.ShapeDtypeStruct((ng, emb), jnp.float32),
    )(z, P1, pb1, gamma, beta, P2, pb2)


def kernel(x, edge_index, batch, roi_scaler, W1, b1, W2, b2, P1, pb1, gamma, beta, P2, pb2):
    assert x.shape[0] == N_NODES
    E = edge_index.shape[1]
    rows_needed = -(-E // LANES)
    rows_total = -(-rows_needed // 256) * 256
    e_pad = rows_total * LANES - E
    src = jnp.concatenate(
        [edge_index[0], jnp.zeros((e_pad,), edge_index.dtype)]).reshape(rows_total, LANES)
    dst = jnp.concatenate(
        [edge_index[1], jnp.full((e_pad,), N_NODES, edge_index.dtype)]).reshape(rows_total, LANES)

    deg2 = _build_deg(rows_total)(dst)
    hs1, dis = _tc1(x, deg2[0].reshape(-1, 1), deg2[1].reshape(-1, 1), roi_scaler, W1)
    scat = _build_scatter(rows_total)
    acc1 = scat(src, dst, hs1)
    hs2 = _tc2(acc1, hs1, dis, b1.reshape(1, HID), W2)
    acc2 = scat(src, dst, hs2)
    z = _tc3(acc2, hs2, dis, b2.reshape(1, HID)).reshape(-1, HID)
    return _tc4(z, P1, pb1.reshape(1, -1), gamma.reshape(1, -1),
                beta.reshape(1, -1), P2, pb2.reshape(1, -1))


# speedup vs baseline: 16.8697x; 1.0141x over previous
import functools

import jax
import jax.numpy as jnp
from jax import lax
from jax.experimental import pallas as pl
from jax.experimental.pallas import tpu as pltpu
from jax.experimental.pallas import tpu_sc as plsc

N_NODES = 50000
HID = 64
QUART = 16
NQ = 4
LANES = 128
JCH = 8
JCS = 10
N_PAD = 51200
ZROWS = 1024
RB = 2000


@functools.lru_cache(maxsize=None)
def _build_deg(rows_total):
    rows_per_core = rows_total // 2
    rows_per_tile = rows_per_core // 16
    n_chunks = rows_per_tile // JCH
    zc = N_PAD // 16
    mesh = plsc.VectorSubcoreMesh(core_axis_name="c", subcore_axis_name="s")

    @functools.partial(
        pl.kernel,
        mesh=mesh,
        out_type=jax.ShapeDtypeStruct((2, N_PAD), jnp.float32),
        compiler_params=pltpu.CompilerParams(use_tc_tiling_on_sc=False),
        scratch_types=[
            pltpu.VMEM((JCH, LANES), jnp.int32),
            pltpu.VMEM((LANES,), jnp.float32),
            pltpu.VMEM((zc,), jnp.float32),
            pltpu.VMEM_SHARED((N_PAD,), jnp.float32),
        ],
    )
    def deg_kernel(dst_hbm, out_hbm, didx, ones_v, zbuf, acc):
        c = lax.axis_index("c")
        s = lax.axis_index("s")
        one16 = jnp.ones((16,), jnp.float32)
        z16 = jnp.zeros((16,), jnp.float32)
        for i in range(LANES // 16):
            ones_v[pl.ds(i * 16, 16)] = one16

        def zr(i, carry):
            zbuf[pl.ds(i * 16, 16)] = z16
            return carry

        lax.fori_loop(0, zc // 16, zr, 0)
        pltpu.sync_copy(zbuf, acc.at[pl.ds(s * zc, zc)])
        plsc.subcore_barrier()

        def chunk(g, carry):
            rb = c * rows_per_core + s * rows_per_tile + g * JCH
            pltpu.sync_copy(dst_hbm.at[pl.ds(rb, JCH)], didx)
            for j in range(JCH):
                pltpu.sync_copy(ones_v, acc.at[didx.at[j]], add=True)
            return carry

        lax.fori_loop(0, n_chunks, chunk, 0)
        plsc.subcore_barrier()
        pltpu.sync_copy(acc.at[pl.ds(s * zc, zc)], out_hbm.at[c].at[pl.ds(s * zc, zc)])

    return deg_kernel


@functools.lru_cache(maxsize=None)
def _build_scatter(rows_total):
    rows_per_tile = rows_total // 16
    n_chunks = rows_per_tile // JCS
    zc = N_PAD // 16
    mesh = plsc.VectorSubcoreMesh(core_axis_name="c", subcore_axis_name="s")

    @functools.partial(
        pl.kernel,
        mesh=mesh,
        out_type=jax.ShapeDtypeStruct((NQ, N_PAD, QUART), jnp.float32),
        compiler_params=pltpu.CompilerParams(use_tc_tiling_on_sc=False),
        scratch_types=[
            pltpu.VMEM((2, JCS, LANES), jnp.int32),
            pltpu.VMEM((2, JCS, LANES), jnp.int32),
            pltpu.VMEM((2, JCS, LANES, QUART), jnp.float32),
            pltpu.VMEM((ZROWS, QUART), jnp.float32),
            pltpu.VMEM_SHARED((N_PAD, QUART), jnp.float32),
            pltpu.SemaphoreType.DMA,
            pltpu.SemaphoreType.DMA,
            pltpu.SemaphoreType.DMA,
        ],
    )
    def scat_kernel(src_hbm, dst_hbm, hs_hbm, out_hbm, sidx, didx, rows, zbuf, acc,
                    gsem, ssem, isem):
        c = lax.axis_index("c")
        s = lax.axis_index("s")
        z16 = jnp.zeros((16,), jnp.float32)

        def zrow(i, carry):
            zbuf[i, pl.ds(0, 16)] = z16
            return carry

        lax.fori_loop(0, ZROWS, zrow, 0)
        zbase = s * zc
        tbase = s * rows_per_tile

        def fire_idx(g, sl):
            pltpu.async_copy(src_hbm.at[pl.ds(tbase + g * JCS, JCS)], sidx.at[sl], isem)
            pltpu.async_copy(dst_hbm.at[pl.ds(tbase + g * JCS, JCS)], didx.at[sl], isem)

        def wait_idx(g, sl):
            pltpu.make_async_copy(
                src_hbm.at[pl.ds(tbase + g * JCS, JCS)], sidx.at[sl], isem).wait()
            pltpu.make_async_copy(
                dst_hbm.at[pl.ds(tbase + g * JCS, JCS)], didx.at[sl], isem).wait()

        for p in range(2):
            q = 2 * c + p
            for k in range(zc // ZROWS):
                pltpu.sync_copy(zbuf, acc.at[pl.ds(zbase + k * ZROWS, ZROWS)])
            rem = zc % ZROWS
            if rem:
                pltpu.sync_copy(zbuf.at[pl.ds(0, rem)],
                                acc.at[pl.ds(zbase + (zc // ZROWS) * ZROWS, rem)])
            plsc.subcore_barrier()

            def fire_g(sl):
                for j in range(JCS):
                    pltpu.async_copy(hs_hbm.at[q].at[sidx.at[sl].at[j]],
                                     rows.at[sl].at[j], gsem)

            def wait_g(sl):
                for j in range(JCS):
                    pltpu.make_async_copy(hs_hbm.at[q].at[sidx.at[sl].at[j]],
                                          rows.at[sl].at[j], gsem).wait()

            def fire_s(sl):
                for j in range(JCS):
                    pltpu.async_copy(rows.at[sl].at[j], acc.at[didx.at[sl].at[j]],
                                     ssem, add=True)

            def wait_s(sl):
                for j in range(JCS):
                    pltpu.make_async_copy(rows.at[sl].at[j], acc.at[didx.at[sl].at[j]],
                                          ssem).wait()

            fire_idx(0, 0)
            wait_idx(0, 0)
            fire_g(0)

            def chunk(g, carry):
                b = g % 2
                nb = 1 - b

                @pl.when(g >= 1)
                def _():
                    wait_s(nb)

                @pl.when(g + 1 < n_chunks)
                def _():
                    fire_idx(g + 1, nb)

                wait_g(b)
                fire_s(b)

                @pl.when(g + 1 < n_chunks)
                def _():
                    wait_idx(g + 1, nb)
                    fire_g(nb)

                return carry

            lax.fori_loop(0, n_chunks, chunk, 0)
            wait_s((n_chunks - 1) % 2)
            plsc.subcore_barrier()
            pltpu.sync_copy(acc.at[pl.ds(zbase, zc)], out_hbm.at[q].at[pl.ds(zbase, zc)])

    return scat_kernel


def _tc1_body(x_ref, dga_ref, dgb_ref, roi_ref, w_ref, hs_ref, dis_ref):
    deg = dga_ref[...] + dgb_ref[...] + 1.0
    dis = lax.rsqrt(deg)
    s = jnp.tile(roi_ref[...], (RB // 100, 1))
    h = jnp.dot(x_ref[...] * s, w_ref[...], preferred_element_type=jnp.float32)
    hs = h * dis
    for q in range(NQ):
        hs_ref[q] = hs[:, q * QUART:(q + 1) * QUART]
    dis_ref[...] = dis


def _tc1(x, dga, dgb, roi, W1):
    nb = N_NODES // RB
    return pl.pallas_call(
        _tc1_body,
        grid=(nb,),
        in_specs=[
            pl.BlockSpec((RB, 16), lambda i: (i, 0)),
            pl.BlockSpec((RB, 1), lambda i: (i, 0)),
            pl.BlockSpec((RB, 1), lambda i: (i, 0)),
            pl.BlockSpec((100, 16), lambda i: (0, 0)),
            pl.BlockSpec((16, HID), lambda i: (0, 0)),
        ],
        out_specs=[
            pl.BlockSpec((NQ, RB, QUART), lambda i: (0, i, 0)),
            pl.BlockSpec((RB, 1), lambda i: (i, 0)),
        ],
        out_shape=[
            jax.ShapeDtypeStruct((NQ, N_NODES, QUART), jnp.float32),
            jax.ShapeDtypeStruct((N_NODES, 1), jnp.float32),
        ],
    )(x, dga, dgb, roi, W1)


def _tc2_body(acc_ref, hs_ref, dis_ref, b1_ref, w2_ref, hs2_ref):
    a = jnp.concatenate(
        [acc_ref[q] + hs_ref[q] for q in range(NQ)], axis=1)
    dis = dis_ref[...]
    h = a * dis + b1_ref[...]
    h = jnp.where(h >= 0, h, 0.2 * h)
    g = jnp.dot(h, w2_ref[...], preferred_element_type=jnp.float32)
    gs = g * dis
    for q in range(NQ):
        hs2_ref[q] = gs[:, q * QUART:(q + 1) * QUART]


def _tc2(acc1, hs1, dis, b1, W2):
    nb = N_NODES // RB
    return pl.pallas_call(
        _tc2_body,
        grid=(nb,),
        in_specs=[
            pl.BlockSpec((NQ, RB, QUART), lambda i: (0, i, 0)),
            pl.BlockSpec((NQ, RB, QUART), lambda i: (0, i, 0)),
            pl.BlockSpec((RB, 1), lambda i: (i, 0)),
            pl.BlockSpec((1, HID), lambda i: (0, 0)),
            pl.BlockSpec((HID, HID), lambda i: (0, 0)),
        ],
        out_specs=pl.BlockSpec((NQ, RB, QUART), lambda i: (0, i, 0)),
        out_shape=jax.ShapeDtypeStruct((NQ, N_NODES, QUART), jnp.float32),
    )(acc1, hs1, dis, b1, W2)


def _tc3_body(acc_ref, hs_ref, dis_ref, b2_ref, z_ref):
    a = jnp.concatenate(
        [acc_ref[q] + hs_ref[q] for q in range(NQ)], axis=1)
    h = a * dis_ref[...] + b2_ref[...]
    h = jnp.where(h >= 0, h, 0.2 * h)
    z_ref[0] = jnp.max(h.reshape(RB // 100, 100, HID), axis=1)


def _tc3(acc2, hs2, dis, b2):
    nb = N_NODES // RB
    return pl.pallas_call(
        _tc3_body,
        grid=(nb,),
        in_specs=[
            pl.BlockSpec((NQ, RB, QUART), lambda i: (0, i, 0)),
            pl.BlockSpec((NQ, RB, QUART), lambda i: (0, i, 0)),
            pl.BlockSpec((RB, 1), lambda i: (i, 0)),
            pl.BlockSpec((1, HID), lambda i: (0, 0)),
        ],
        out_specs=pl.BlockSpec((1, RB // 100, HID), lambda i: (i, 0, 0)),
        out_shape=jax.ShapeDtypeStruct((nb, RB // 100, HID), jnp.float32),
    )(acc2, hs2, dis, b2)


def _tc4_body(z_ref, p1_ref, pb1_ref, g_ref, b_ref, p2_ref, pb2_ref, o_ref):
    z = jnp.dot(z_ref[...], p1_ref[...], preferred_element_type=jnp.float32) + pb1_ref[...]
    mean = jnp.mean(z, axis=0, keepdims=True)
    var = jnp.mean((z - mean) ** 2, axis=0, keepdims=True)
    zn = (z - mean) * lax.rsqrt(var + 1e-5) * g_ref[...] + b_ref[...]
    zn = jnp.where(zn >= 0, zn, 0.2 * zn)
    z2 = jnp.dot(zn, p2_ref[...], preferred_element_type=jnp.float32) + pb2_ref[...]
    nrm = jnp.sqrt(jnp.sum(z2 * z2, axis=1, keepdims=True))
    o_ref[...] = z2 / jnp.maximum(nrm, 1e-12)


def _tc4(z, P1, pb1, gamma, beta, P2, pb2):
    ng, emb = z.shape[0], P2.shape[1]
    return pl.pallas_call(
        _tc4_body,
        out_shape=jax.ShapeDtypeStruct((ng, emb), jnp.float32),
    )(z, P1, pb1, gamma, beta, P2, pb2)


def kernel(x, edge_index, batch, roi_scaler, W1, b1, W2, b2, P1, pb1, gamma, beta, P2, pb2):
    assert x.shape[0] == N_NODES
    E = edge_index.shape[1]
    rows_needed = -(-E // LANES)
    rows_total = -(-rows_needed // 256) * 256
    e_pad = rows_total * LANES - E
    src = jnp.concatenate(
        [edge_index[0], jnp.zeros((e_pad,), edge_index.dtype)]).reshape(rows_total, LANES)
    dst = jnp.concatenate(
        [edge_index[1], jnp.full((e_pad,), N_NODES, edge_index.dtype)]).reshape(rows_total, LANES)

    deg2 = _build_deg(rows_total)(dst)
    hs1, dis = _tc1(x, deg2[0].reshape(-1, 1), deg2[1].reshape(-1, 1), roi_scaler, W1)
    scat = _build_scatter(rows_total)
    acc1 = scat(src, dst, hs1)
    hs2 = _tc2(acc1, hs1, dis, b1.reshape(1, HID), W2)
    acc2 = scat(src, dst, hs2)
    z = _tc3(acc2, hs2, dis, b2.reshape(1, HID)).reshape(-1, HID)
    return _tc4(z, P1, pb1.reshape(1, -1), gamma.reshape(1, -1),
                beta.reshape(1, -1), P2, pb2.reshape(1, -1))
